# Initial kernel scaffold; baseline (speedup 1.0000x reference)
import jax
import jax.numpy as jnp
from jax.experimental import pallas as pl


def kernel(queries, keys, k):
    raise NotImplementedError("write your pallas kernel here")


# speedup vs baseline: 5.5177x; 5.5177x over previous
import functools

import jax
import jax.numpy as jnp
from jax import lax
from jax.experimental import pallas as pl
from jax.experimental.pallas import tpu as pltpu
from jax.experimental.pallas import tpu_sc as plsc

Q = 4096
N = 16384
DIM = 128
KTOP = 16
G = 32
NG = N // G
CHUNK = 128
NCH = N // CHUNK
QB = 256
KB = 4096
QBLK = Q // QB
KBLK = N // KB
GPB = KB // G

SC_NC = 2
SC_NS = 16
SC_NW = SC_NC * SC_NS
IDX_ROWS = (Q * KTOP) // 128
ROWS_PER_W = IDX_ROWS // SC_NW
SC_PASS = 4

_INTERPRET = False


def _tc1_body(q_ref, k_ref, d_ref, m_ref):
    q = q_ref[...]
    kv = k_ref[...]
    qs = jnp.sum(q * q, axis=1, keepdims=True)
    ks = jnp.sum(kv * kv, axis=1)[None, :]
    cross = lax.dot_general(q, kv, (((1,), (1,)), ((), ())),
                            preferred_element_type=jnp.float32)
    d = qs - 2.0 * cross + ks
    d_ref[...] = d
    mins = [jnp.min(d[:, g * G:(g + 1) * G], axis=1, keepdims=True)
            for g in range(GPB)]
    m_ref[...] = jnp.concatenate(mins, axis=1)


def _tc1(queries, keys):
    return pl.pallas_call(
        _tc1_body,
        grid=(QBLK, KBLK),
        in_specs=[
            pl.BlockSpec((QB, DIM), lambda i, j: (i, 0)),
            pl.BlockSpec((KB, DIM), lambda i, j: (j, 0)),
        ],
        out_specs=[
            pl.BlockSpec((QB, KB), lambda i, j: (i, j)),
            pl.BlockSpec((QB, GPB), lambda i, j: (i, j)),
        ],
        out_shape=[
            jax.ShapeDtypeStruct((Q, N), jnp.float32),
            jax.ShapeDtypeStruct((Q, NG), jnp.float32),
        ],
        interpret=_INTERPRET,
    )(queries, keys)


def _tc1b_body(m_ref, g_ref, r_ref):
    i = pl.program_id(0)
    m = m_ref[...]
    git = lax.broadcasted_iota(jnp.int32, (QB, NG), 1)
    cols = []
    for _ in range(KTOP):
        mn = jnp.min(m, axis=1, keepdims=True)
        eq = m == mn
        si = jnp.min(jnp.where(eq, git, 1 << 30), axis=1, keepdims=True)
        cols.append(si)
        m = jnp.where(git == si, jnp.float32(jnp.inf), m)
    gid = jnp.concatenate(cols, axis=1)
    g_ref[...] = gid
    qg = lax.broadcasted_iota(jnp.int32, (QB, KTOP), 0) + i * QB
    r_ref[...] = qg * NCH + gid // (CHUNK // G)


def _tc1b(m):
    return pl.pallas_call(
        _tc1b_body,
        grid=(QBLK,),
        in_specs=[pl.BlockSpec((QB, NG), lambda i: (i, 0))],
        out_specs=[
            pl.BlockSpec((QB, KTOP), lambda i: (i, 0)),
            pl.BlockSpec((QB, KTOP), lambda i: (i, 0)),
        ],
        out_shape=[
            jax.ShapeDtypeStruct((Q, KTOP), jnp.int32),
            jax.ShapeDtypeStruct((Q, KTOP), jnp.int32),
        ],
        interpret=_INTERPRET,
    )(m)


def _gather_candidates(table, idx2):
    mesh = plsc.VectorSubcoreMesh(core_axis_name="c", subcore_axis_name="s")

    @functools.partial(
        pl.kernel,
        out_type=jax.ShapeDtypeStruct((IDX_ROWS, 128, CHUNK), jnp.float32),
        mesh=mesh,
        scratch_types=[
            pltpu.VMEM((ROWS_PER_W, 128), jnp.int32),
            pltpu.VMEM((SC_PASS, 128, CHUNK), jnp.float32),
            pltpu.SemaphoreType.DMA,
        ],
    )
    def sc_gather(table_hbm, idx_hbm, out_hbm, idx_v, rows_v, sem):
        wid = lax.axis_index("s") * SC_NC + lax.axis_index("c")
        base = wid * ROWS_PER_W
        pltpu.sync_copy(idx_hbm.at[pl.ds(base, ROWS_PER_W)], idx_v)
        for p in range(ROWS_PER_W // SC_PASS):
            handles = [
                pltpu.async_copy(table_hbm.at[idx_v.at[p * SC_PASS + c]],
                                 rows_v.at[c], sem)
                for c in range(SC_PASS)
            ]
            for h in handles:
                h.wait()
            pltpu.sync_copy(
                rows_v, out_hbm.at[pl.ds(base + p * SC_PASS, SC_PASS)])

    return sc_gather(table, idx2)


def _tc3_body(c_ref, g_ref, d_ref, i_ref):
    chunks = c_ref[...]
    gid = g_ref[...]
    off = lax.broadcasted_iota(jnp.int32, (QB, G), 1)
    vparts, iparts = [], []
    for s in range(KTOP):
        ch = chunks[:, s * CHUNK:(s + 1) * CHUNK]
        gs = gid[:, s:s + 1]
        sub = gs % (CHUNK // G)
        v = ch[:, 0:G]
        for t in range(1, CHUNK // G):
            v = jnp.where(sub == t, ch[:, t * G:(t + 1) * G], v)
        vparts.append(v)
        iparts.append(gs * G + off)
    vals = jnp.concatenate(vparts, axis=1)
    cidx = jnp.concatenate(iparts, axis=1)
    dl, il = [], []
    for _ in range(KTOP):
        m = jnp.min(vals, axis=1, keepdims=True)
        eq = vals == m
        si = jnp.min(jnp.where(eq, cidx, 1 << 30), axis=1, keepdims=True)
        dl.append(m)
        il.append(si)
        vals = jnp.where(cidx == si, jnp.float32(jnp.inf), vals)
    d_ref[...] = jnp.concatenate(dl, axis=1)
    i_ref[...] = jnp.concatenate(il, axis=1)


def _tc3(cand, gid):
    return pl.pallas_call(
        _tc3_body,
        grid=(QBLK,),
        in_specs=[
            pl.BlockSpec((QB, KTOP * CHUNK), lambda i: (i, 0)),
            pl.BlockSpec((QB, KTOP), lambda i: (i, 0)),
        ],
        out_specs=[
            pl.BlockSpec((QB, KTOP), lambda i: (i, 0)),
            pl.BlockSpec((QB, KTOP), lambda i: (i, 0)),
        ],
        out_shape=[
            jax.ShapeDtypeStruct((Q, KTOP), jnp.float32),
            jax.ShapeDtypeStruct((Q, KTOP), jnp.int32),
        ],
        interpret=_INTERPRET,
    )(cand, gid)


def kernel(queries, keys, k):
    dmat, m = _tc1(queries, keys)
    gid, rowid = _tc1b(m)
    cand3 = _gather_candidates(
        dmat.reshape(Q * NCH, CHUNK), rowid.reshape(IDX_ROWS, 128))
    dists, idx = _tc3(cand3.reshape(Q, KTOP * CHUNK), gid)
    shift = (jnp.asarray(k) - KTOP).astype(jnp.float32)
    return (dists + shift, idx)


# speedup vs baseline: 6.5955x; 1.1953x over previous
import functools

import jax
import jax.numpy as jnp
from jax import lax
from jax.experimental import pallas as pl
from jax.experimental.pallas import tpu as pltpu
from jax.experimental.pallas import tpu_sc as plsc

Q = 4096
N = 16384
DIM = 128
KTOP = 16
G = 128
NG = N // G
QB = 256
KB = 4096
QBLK = Q // QB
KBLK = N // KB
GPB = KB // G

SC_NC = 2
SC_NS = 16
SC_NW = SC_NC * SC_NS
IDX_ROWS = (Q * KTOP) // 128
ROWS_PER_W = IDX_ROWS // SC_NW
SC_PASS = 4

_INTERPRET = False


def _tc1_body(q_ref, k_ref, d_ref, m_ref):
    q = q_ref[...]
    kv = k_ref[...]
    qs = jnp.sum(q * q, axis=1, keepdims=True)
    ks = jnp.sum(kv * kv, axis=1)[None, :]
    cross = lax.dot_general(q, kv, (((1,), (1,)), ((), ())),
                            preferred_element_type=jnp.float32)
    d = qs - 2.0 * cross + ks
    d_ref[...] = d
    j = pl.program_id(1)
    lane = lax.broadcasted_iota(jnp.int32, (QB, NG), 1)
    placed = jnp.full((QB, NG), jnp.inf, jnp.float32)
    for g in range(GPB):
        mg = jnp.min(d[:, g * G:(g + 1) * G], axis=1, keepdims=True)
        placed = jnp.where(lane == j * GPB + g, mg, placed)
    prev = jnp.where(j == 0, jnp.float32(jnp.inf), m_ref[...])
    m_ref[...] = jnp.minimum(prev, placed)


def _tc1(queries, keys):
    return pl.pallas_call(
        _tc1_body,
        grid=(QBLK, KBLK),
        in_specs=[
            pl.BlockSpec((QB, DIM), lambda i, j: (i, 0)),
            pl.BlockSpec((KB, DIM), lambda i, j: (j, 0)),
        ],
        out_specs=[
            pl.BlockSpec((QB, KB), lambda i, j: (i, j)),
            pl.BlockSpec((QB, NG), lambda i, j: (i, 0)),
        ],
        out_shape=[
            jax.ShapeDtypeStruct((Q, N), jnp.float32),
            jax.ShapeDtypeStruct((Q, NG), jnp.float32),
        ],
        interpret=_INTERPRET,
    )(queries, keys)


def _tc1b_body(m_ref, g_ref, r_ref):
    i = pl.program_id(0)
    m = m_ref[...]
    git = lax.broadcasted_iota(jnp.int32, (QB, NG), 1)
    cols = []
    for _ in range(KTOP):
        mn = jnp.min(m, axis=1, keepdims=True)
        eq = m == mn
        si = jnp.min(jnp.where(eq, git, 1 << 30), axis=1, keepdims=True)
        cols.append(si)
        m = jnp.where(git == si, jnp.float32(jnp.inf), m)
    gid = jnp.concatenate(cols, axis=1)
    g_ref[...] = gid
    qg = lax.broadcasted_iota(jnp.int32, (QB, KTOP), 0) + i * QB
    r_ref[...] = qg * NG + gid


def _tc1b(m):
    return pl.pallas_call(
        _tc1b_body,
        grid=(QBLK,),
        in_specs=[pl.BlockSpec((QB, NG), lambda i: (i, 0))],
        out_specs=[
            pl.BlockSpec((QB, KTOP), lambda i: (i, 0)),
            pl.BlockSpec((QB, KTOP), lambda i: (i, 0)),
        ],
        out_shape=[
            jax.ShapeDtypeStruct((Q, KTOP), jnp.int32),
            jax.ShapeDtypeStruct((Q, KTOP), jnp.int32),
        ],
        interpret=_INTERPRET,
    )(m)


def _gather_candidates(table, idx2):
    mesh = plsc.VectorSubcoreMesh(core_axis_name="c", subcore_axis_name="s")

    @functools.partial(
        pl.kernel,
        out_type=jax.ShapeDtypeStruct((IDX_ROWS, 128, G), jnp.float32),
        mesh=mesh,
        scratch_types=[
            pltpu.VMEM((ROWS_PER_W, 128), jnp.int32),
            pltpu.VMEM((SC_PASS, 128, G), jnp.float32),
            pltpu.SemaphoreType.DMA,
        ],
    )
    def sc_gather(table_hbm, idx_hbm, out_hbm, idx_v, rows_v, sem):
        wid = lax.axis_index("s") * SC_NC + lax.axis_index("c")
        base = wid * ROWS_PER_W
        pltpu.sync_copy(idx_hbm.at[pl.ds(base, ROWS_PER_W)], idx_v)
        for p in range(ROWS_PER_W // SC_PASS):
            handles = [
                pltpu.async_copy(table_hbm.at[idx_v.at[p * SC_PASS + c]],
                                 rows_v.at[c], sem)
                for c in range(SC_PASS)
            ]
            for h in handles:
                h.wait()
            pltpu.sync_copy(
                rows_v, out_hbm.at[pl.ds(base + p * SC_PASS, SC_PASS)])

    return sc_gather(table, idx2)


def _tc3_body(c_ref, g_ref, d_ref, i_ref):
    vals = c_ref[...]
    gid = g_ref[...]
    off = lax.broadcasted_iota(jnp.int32, (QB, G), 1)
    iparts = [gid[:, s:s + 1] * G + off for s in range(KTOP)]
    cidx = jnp.concatenate(iparts, axis=1)
    dl, il = [], []
    for _ in range(KTOP):
        m = jnp.min(vals, axis=1, keepdims=True)
        eq = vals == m
        si = jnp.min(jnp.where(eq, cidx, 1 << 30), axis=1, keepdims=True)
        dl.append(m)
        il.append(si)
        vals = jnp.where(cidx == si, jnp.float32(jnp.inf), vals)
    d_ref[...] = jnp.concatenate(dl, axis=1)
    i_ref[...] = jnp.concatenate(il, axis=1)


def _tc3(cand, gid):
    return pl.pallas_call(
        _tc3_body,
        grid=(QBLK,),
        in_specs=[
            pl.BlockSpec((QB, KTOP * G), lambda i: (i, 0)),
            pl.BlockSpec((QB, KTOP), lambda i: (i, 0)),
        ],
        out_specs=[
            pl.BlockSpec((QB, KTOP), lambda i: (i, 0)),
            pl.BlockSpec((QB, KTOP), lambda i: (i, 0)),
        ],
        out_shape=[
            jax.ShapeDtypeStruct((Q, KTOP), jnp.float32),
            jax.ShapeDtypeStruct((Q, KTOP), jnp.int32),
        ],
        interpret=_INTERPRET,
    )(cand, gid)


def kernel(queries, keys, k):
    dmat, m = _tc1(queries, keys)
    gid, rowid = _tc1b(m)
    cand3 = _gather_candidates(
        dmat.reshape(Q * NG, G), rowid.reshape(IDX_ROWS, 128))
    dists, idx = _tc3(cand3.reshape(Q, KTOP * G), gid)
    shift = (jnp.asarray(k) - KTOP).astype(jnp.float32)
    return (dists + shift, idx)


# speedup vs baseline: 10.9538x; 1.6608x over previous
import functools

import jax
import jax.numpy as jnp
from jax import lax
from jax.experimental import pallas as pl
from jax.experimental.pallas import tpu as pltpu
from jax.experimental.pallas import tpu_sc as plsc

Q = 4096
N = 16384
DIM = 128
KTOP = 16
G = 128
NG = N // G
QB = 256
KB = 4096
QBLK = Q // QB
KBLK = N // KB
GPB = KB // G

SC_NC = 2
SC_NS = 16
SC_NW = SC_NC * SC_NS
IDX_ROWS = (Q * KTOP) // 128
ROWS_PER_W = IDX_ROWS // SC_NW
SC_PASS = 4

_INTERPRET = False


def _tc1_body(q_ref, k_ref, t_ref, m_ref):
    q = q_ref[...]
    kv = k_ref[...]
    qs = jnp.sum(q * q, axis=1, keepdims=True)
    ks = jnp.sum(kv * kv, axis=1)[None, :]
    cross = lax.dot_general(q, kv, (((1,), (1,)), ((), ())),
                            preferred_element_type=jnp.float32)
    d = qs - 2.0 * cross + ks
    for c in range(GPB):
        t_ref[c] = d[:, c * G:(c + 1) * G]
    j = pl.program_id(1)
    lane = lax.broadcasted_iota(jnp.int32, (QB, NG), 1)
    placed = jnp.full((QB, NG), jnp.inf, jnp.float32)
    for g in range(GPB):
        mg = jnp.min(d[:, g * G:(g + 1) * G], axis=1, keepdims=True)
        placed = jnp.where(lane == j * GPB + g, mg, placed)
    prev = jnp.where(j == 0, jnp.float32(jnp.inf), m_ref[...])
    m_ref[...] = jnp.minimum(prev, placed)


def _tc1(queries, keys):
    return pl.pallas_call(
        _tc1_body,
        grid=(QBLK, KBLK),
        in_specs=[
            pl.BlockSpec((QB, DIM), lambda i, j: (i, 0)),
            pl.BlockSpec((KB, DIM), lambda i, j: (j, 0)),
        ],
        out_specs=[
            pl.BlockSpec((GPB, QB, G), lambda i, j: (j, i, 0)),
            pl.BlockSpec((QB, NG), lambda i, j: (i, 0)),
        ],
        out_shape=[
            jax.ShapeDtypeStruct((NG, Q, G), jnp.float32),
            jax.ShapeDtypeStruct((Q, NG), jnp.float32),
        ],
        interpret=_INTERPRET,
    )(queries, keys)


def _tc1b_body(m_ref, g_ref, r_ref):
    i = pl.program_id(0)
    m = m_ref[...]
    git = lax.broadcasted_iota(jnp.int32, (QB, NG), 1)
    cols = []
    for _ in range(KTOP):
        mn = jnp.min(m, axis=1, keepdims=True)
        eq = m == mn
        si = jnp.min(jnp.where(eq, git, 1 << 30), axis=1, keepdims=True)
        cols.append(si)
        m = jnp.where(git == si, jnp.float32(jnp.inf), m)
    gid = jnp.concatenate(cols, axis=1)
    g_ref[...] = gid
    qg = lax.broadcasted_iota(jnp.int32, (QB, KTOP), 0) + i * QB
    r_ref[...] = gid * Q + qg


def _tc1b(m):
    return pl.pallas_call(
        _tc1b_body,
        grid=(QBLK,),
        in_specs=[pl.BlockSpec((QB, NG), lambda i: (i, 0))],
        out_specs=[
            pl.BlockSpec((QB, KTOP), lambda i: (i, 0)),
            pl.BlockSpec((QB, KTOP), lambda i: (i, 0)),
        ],
        out_shape=[
            jax.ShapeDtypeStruct((Q, KTOP), jnp.int32),
            jax.ShapeDtypeStruct((Q, KTOP), jnp.int32),
        ],
        interpret=_INTERPRET,
    )(m)


def _gather_candidates(table, idx2):
    mesh = plsc.VectorSubcoreMesh(core_axis_name="c", subcore_axis_name="s")

    @functools.partial(
        pl.kernel,
        out_type=jax.ShapeDtypeStruct((IDX_ROWS, 128, G), jnp.float32),
        mesh=mesh,
        scratch_types=[
            pltpu.VMEM((ROWS_PER_W, 128), jnp.int32),
            pltpu.VMEM((SC_PASS, 128, G), jnp.float32),
            pltpu.SemaphoreType.DMA,
        ],
    )
    def sc_gather(table_hbm, idx_hbm, out_hbm, idx_v, rows_v, sem):
        wid = lax.axis_index("s") * SC_NC + lax.axis_index("c")
        base = wid * ROWS_PER_W
        pltpu.sync_copy(idx_hbm.at[pl.ds(base, ROWS_PER_W)], idx_v)
        for p in range(ROWS_PER_W // SC_PASS):
            handles = [
                pltpu.async_copy(table_hbm.at[idx_v.at[p * SC_PASS + c]],
                                 rows_v.at[c], sem)
                for c in range(SC_PASS)
            ]
            for h in handles:
                h.wait()
            pltpu.sync_copy(
                rows_v, out_hbm.at[pl.ds(base + p * SC_PASS, SC_PASS)])

    return sc_gather(table, idx2)


def _tc3_body(c_ref, g_ref, d_ref, i_ref):
    vals = c_ref[...]
    gid = g_ref[...]
    off = lax.broadcasted_iota(jnp.int32, (QB, G), 1)
    iparts = [gid[:, s:s + 1] * G + off for s in range(KTOP)]
    cidx = jnp.concatenate(iparts, axis=1)
    dl, il = [], []
    for _ in range(KTOP):
        m = jnp.min(vals, axis=1, keepdims=True)
        eq = vals == m
        si = jnp.min(jnp.where(eq, cidx, 1 << 30), axis=1, keepdims=True)
        dl.append(m)
        il.append(si)
        vals = jnp.where(cidx == si, jnp.float32(jnp.inf), vals)
    d_ref[...] = jnp.concatenate(dl, axis=1)
    i_ref[...] = jnp.concatenate(il, axis=1)


def _tc3(cand, gid):
    return pl.pallas_call(
        _tc3_body,
        grid=(QBLK,),
        in_specs=[
            pl.BlockSpec((QB, KTOP * G), lambda i: (i, 0)),
            pl.BlockSpec((QB, KTOP), lambda i: (i, 0)),
        ],
        out_specs=[
            pl.BlockSpec((QB, KTOP), lambda i: (i, 0)),
            pl.BlockSpec((QB, KTOP), lambda i: (i, 0)),
        ],
        out_shape=[
            jax.ShapeDtypeStruct((Q, KTOP), jnp.float32),
            jax.ShapeDtypeStruct((Q, KTOP), jnp.int32),
        ],
        interpret=_INTERPRET,
    )(cand, gid)


def kernel(queries, keys, k):
    table, m = _tc1(queries, keys)
    gid, rowid = _tc1b(m)
    cand3 = _gather_candidates(
        table.reshape(NG * Q, G), rowid.reshape(IDX_ROWS, 128))
    dists, idx = _tc3(cand3.reshape(Q, KTOP * G), gid)
    shift = (jnp.asarray(k) - KTOP).astype(jnp.float32)
    return (dists + shift, idx)


# speedup vs baseline: 11.5028x; 1.0501x over previous
import functools

import jax
import jax.numpy as jnp
from jax import lax
from jax.experimental import pallas as pl
from jax.experimental.pallas import tpu as pltpu
from jax.experimental.pallas import tpu_sc as plsc

Q = 4096
N = 16384
DIM = 128
KTOP = 16
G = 128
NG = N // G
QB = 256
KB = 4096
QBLK = Q // QB
KBLK = N // KB
GPB = KB // G

SC_NC = 2
SC_NS = 16
SC_NW = SC_NC * SC_NS
IDX_ROWS = (Q * KTOP) // 128
ROWS_PER_W = IDX_ROWS // SC_NW
SC_PASS = 4

_INTERPRET = False


def _tc1_body(q_ref, k_ref, t_ref, m_ref):
    j = pl.program_id(0)
    i = pl.program_id(1)
    q = q_ref[...]
    kv = k_ref[...]
    qs = jnp.sum(q * q, axis=1, keepdims=True)
    ks = jnp.sum(kv * kv, axis=1)[None, :]
    cross = lax.dot_general(q, kv, (((1,), (1,)), ((), ())),
                            preferred_element_type=jnp.float32)
    d = qs - 2.0 * cross + ks
    for c in range(GPB):
        t_ref[c] = d[:, c * G:(c + 1) * G]
    lane = lax.broadcasted_iota(jnp.int32, (QB, NG), 1)
    placed = jnp.full((QB, NG), jnp.inf, jnp.float32)
    for g in range(GPB):
        mg = jnp.min(d[:, g * G:(g + 1) * G], axis=1, keepdims=True)
        placed = jnp.where(lane == j * GPB + g, mg, placed)
    slab = m_ref[pl.ds(i * QB, QB), :]
    prev = jnp.where(j == 0, jnp.float32(jnp.inf), slab)
    m_ref[pl.ds(i * QB, QB), :] = jnp.minimum(prev, placed)


def _tc1(queries, keys):
    return pl.pallas_call(
        _tc1_body,
        grid=(KBLK, QBLK),
        in_specs=[
            pl.BlockSpec((QB, DIM), lambda j, i: (i, 0)),
            pl.BlockSpec((KB, DIM), lambda j, i: (j, 0)),
        ],
        out_specs=[
            pl.BlockSpec((GPB, QB, G), lambda j, i: (j, i, 0)),
            pl.BlockSpec((Q, NG), lambda j, i: (0, 0)),
        ],
        out_shape=[
            jax.ShapeDtypeStruct((NG, Q, G), jnp.float32),
            jax.ShapeDtypeStruct((Q, NG), jnp.float32),
        ],
        interpret=_INTERPRET,
    )(queries, keys)


def _tc1b_body(m_ref, g_ref, r_ref):
    i = pl.program_id(0)
    m = m_ref[...]
    git = lax.broadcasted_iota(jnp.int32, (QB, NG), 1)
    cols = []
    for _ in range(KTOP):
        mn = jnp.min(m, axis=1, keepdims=True)
        eq = m == mn
        si = jnp.min(jnp.where(eq, git, 1 << 30), axis=1, keepdims=True)
        cols.append(si)
        m = jnp.where(git == si, jnp.float32(jnp.inf), m)
    gid = jnp.concatenate(cols, axis=1)
    g_ref[...] = gid
    qg = lax.broadcasted_iota(jnp.int32, (QB, KTOP), 0) + i * QB
    r_ref[...] = gid * Q + qg


def _tc1b(m):
    return pl.pallas_call(
        _tc1b_body,
        grid=(QBLK,),
        in_specs=[pl.BlockSpec((QB, NG), lambda i: (i, 0))],
        out_specs=[
            pl.BlockSpec((QB, KTOP), lambda i: (i, 0)),
            pl.BlockSpec((QB, KTOP), lambda i: (i, 0)),
        ],
        out_shape=[
            jax.ShapeDtypeStruct((Q, KTOP), jnp.int32),
            jax.ShapeDtypeStruct((Q, KTOP), jnp.int32),
        ],
        interpret=_INTERPRET,
    )(m)


def _gather_candidates(table, idx2):
    mesh = plsc.VectorSubcoreMesh(core_axis_name="c", subcore_axis_name="s")

    @functools.partial(
        pl.kernel,
        out_type=jax.ShapeDtypeStruct((IDX_ROWS, 128, G), jnp.float32),
        mesh=mesh,
        scratch_types=[
            pltpu.VMEM((ROWS_PER_W, 128), jnp.int32),
            pltpu.VMEM((SC_PASS, 128, G), jnp.float32),
            pltpu.SemaphoreType.DMA,
        ],
    )
    def sc_gather(table_hbm, idx_hbm, out_hbm, idx_v, rows_v, sem):
        wid = lax.axis_index("s") * SC_NC + lax.axis_index("c")
        base = wid * ROWS_PER_W
        pltpu.sync_copy(idx_hbm.at[pl.ds(base, ROWS_PER_W)], idx_v)
        for p in range(ROWS_PER_W // SC_PASS):
            handles = [
                pltpu.async_copy(table_hbm.at[idx_v.at[p * SC_PASS + c]],
                                 rows_v.at[c], sem)
                for c in range(SC_PASS)
            ]
            for h in handles:
                h.wait()
            pltpu.sync_copy(
                rows_v, out_hbm.at[pl.ds(base + p * SC_PASS, SC_PASS)])

    return sc_gather(table, idx2)


def _tc3_body(c_ref, g_ref, d_ref, i_ref):
    vals = c_ref[...]
    gid = g_ref[...]
    off = lax.broadcasted_iota(jnp.int32, (QB, G), 1)
    iparts = [gid[:, s:s + 1] * G + off for s in range(KTOP)]
    cidx = jnp.concatenate(iparts, axis=1)
    dl, il = [], []
    for _ in range(KTOP):
        m = jnp.min(vals, axis=1, keepdims=True)
        eq = vals == m
        si = jnp.min(jnp.where(eq, cidx, 1 << 30), axis=1, keepdims=True)
        dl.append(m)
        il.append(si)
        vals = jnp.where(cidx == si, jnp.float32(jnp.inf), vals)
    d_ref[...] = jnp.concatenate(dl, axis=1)
    i_ref[...] = jnp.concatenate(il, axis=1)


def _tc3(cand, gid):
    return pl.pallas_call(
        _tc3_body,
        grid=(QBLK,),
        in_specs=[
            pl.BlockSpec((QB, KTOP * G), lambda i: (i, 0)),
            pl.BlockSpec((QB, KTOP), lambda i: (i, 0)),
        ],
        out_specs=[
            pl.BlockSpec((QB, KTOP), lambda i: (i, 0)),
            pl.BlockSpec((QB, KTOP), lambda i: (i, 0)),
        ],
        out_shape=[
            jax.ShapeDtypeStruct((Q, KTOP), jnp.float32),
            jax.ShapeDtypeStruct((Q, KTOP), jnp.int32),
        ],
        interpret=_INTERPRET,
    )(cand, gid)


def kernel(queries, keys, k):
    table, m = _tc1(queries, keys)
    gid, rowid = _tc1b(m)
    cand3 = _gather_candidates(
        table.reshape(NG * Q, G), rowid.reshape(IDX_ROWS, 128))
    dists, idx = _tc3(cand3.reshape(Q, KTOP * G), gid)
    shift = (jnp.asarray(k) - KTOP).astype(jnp.float32)
    return (dists + shift, idx)


# speedup vs baseline: 11.5329x; 1.0026x over previous
import functools

import jax
import jax.numpy as jnp
from jax import lax
from jax.experimental import pallas as pl
from jax.experimental.pallas import tpu as pltpu
from jax.experimental.pallas import tpu_sc as plsc

Q = 4096
N = 16384
DIM = 128
KTOP = 16
G = 128
NG = N // G
QB = 256
KB = 4096
QBLK = Q // QB
KBLK = N // KB
GPB = KB // G

SC_NC = 2
SC_NS = 16
SC_NW = SC_NC * SC_NS
IDX_ROWS = (Q * KTOP) // 128
ROWS_PER_W = IDX_ROWS // SC_NW
SC_PASS = 4

_INTERPRET = False


def _tc1_body(q_ref, k_ref, t_ref, g_ref, r_ref, scratch_ref):
    j = pl.program_id(0)
    i = pl.program_id(1)
    q = q_ref[...]
    kv = k_ref[...]
    qs = jnp.sum(q * q, axis=1, keepdims=True)
    ks = jnp.sum(kv * kv, axis=1)[None, :]
    cross = lax.dot_general(q, kv, (((1,), (1,)), ((), ())),
                            preferred_element_type=jnp.float32)
    d = qs - 2.0 * cross + ks
    for c in range(GPB):
        t_ref[c] = d[:, c * G:(c + 1) * G]
    lane = lax.broadcasted_iota(jnp.int32, (QB, NG), 1)
    placed = jnp.full((QB, NG), jnp.inf, jnp.float32)
    for g in range(GPB):
        mg = jnp.min(d[:, g * G:(g + 1) * G], axis=1, keepdims=True)
        placed = jnp.where(lane == j * GPB + g, mg, placed)
    slab = scratch_ref[pl.ds(i * QB, QB), :]
    prev = jnp.where(j == 0, jnp.float32(jnp.inf), slab)
    mcur = jnp.minimum(prev, placed)
    scratch_ref[pl.ds(i * QB, QB), :] = mcur

    @pl.when(j == KBLK - 1)
    def _select_groups():
        m = mcur
        git = lax.broadcasted_iota(jnp.int32, (QB, NG), 1)
        cols = []
        for _ in range(KTOP):
            mn = jnp.min(m, axis=1, keepdims=True)
            eq = m == mn
            si = jnp.min(jnp.where(eq, git, 1 << 30), axis=1, keepdims=True)
            cols.append(si)
            m = jnp.where(git == si, jnp.float32(jnp.inf), m)
        gid = jnp.concatenate(cols, axis=1)
        g_ref[...] = gid
        qg = lax.broadcasted_iota(jnp.int32, (QB, KTOP), 0) + i * QB
        r_ref[...] = gid * Q + qg


def _tc1(queries, keys):
    return pl.pallas_call(
        _tc1_body,
        grid=(KBLK, QBLK),
        in_specs=[
            pl.BlockSpec((QB, DIM), lambda j, i: (i, 0)),
            pl.BlockSpec((KB, DIM), lambda j, i: (j, 0)),
        ],
        out_specs=[
            pl.BlockSpec((GPB, QB, G), lambda j, i: (j, i, 0)),
            pl.BlockSpec((QB, KTOP), lambda j, i: (i, 0)),
            pl.BlockSpec((QB, KTOP), lambda j, i: (i, 0)),
        ],
        out_shape=[
            jax.ShapeDtypeStruct((NG, Q, G), jnp.float32),
            jax.ShapeDtypeStruct((Q, KTOP), jnp.int32),
            jax.ShapeDtypeStruct((Q, KTOP), jnp.int32),
        ],
        scratch_shapes=[pltpu.VMEM((Q, NG), jnp.float32)],
        interpret=_INTERPRET,
    )(queries, keys)


def _gather_candidates(table, idx2):
    mesh = plsc.VectorSubcoreMesh(core_axis_name="c", subcore_axis_name="s")

    @functools.partial(
        pl.kernel,
        out_type=jax.ShapeDtypeStruct((IDX_ROWS, 128, G), jnp.float32),
        mesh=mesh,
        scratch_types=[
            pltpu.VMEM((ROWS_PER_W, 128), jnp.int32),
            pltpu.VMEM((SC_PASS, 128, G), jnp.float32),
            pltpu.SemaphoreType.DMA,
        ],
    )
    def sc_gather(table_hbm, idx_hbm, out_hbm, idx_v, rows_v, sem):
        wid = lax.axis_index("s") * SC_NC + lax.axis_index("c")
        base = wid * ROWS_PER_W
        pltpu.sync_copy(idx_hbm.at[pl.ds(base, ROWS_PER_W)], idx_v)
        for p in range(ROWS_PER_W // SC_PASS):
            handles = [
                pltpu.async_copy(table_hbm.at[idx_v.at[p * SC_PASS + c]],
                                 rows_v.at[c], sem)
                for c in range(SC_PASS)
            ]
            for h in handles:
                h.wait()
            pltpu.sync_copy(
                rows_v, out_hbm.at[pl.ds(base + p * SC_PASS, SC_PASS)])

    return sc_gather(table, idx2)


def _tc3_body(c_ref, g_ref, d_ref, i_ref):
    vals = c_ref[...]
    gid = g_ref[...]
    off = lax.broadcasted_iota(jnp.int32, (QB, G), 1)
    iparts = [gid[:, s:s + 1] * G + off for s in range(KTOP)]
    cidx = jnp.concatenate(iparts, axis=1)
    dl, il = [], []
    for _ in range(KTOP):
        m = jnp.min(vals, axis=1, keepdims=True)
        eq = vals == m
        si = jnp.min(jnp.where(eq, cidx, 1 << 30), axis=1, keepdims=True)
        dl.append(m)
        il.append(si)
        vals = jnp.where(cidx == si, jnp.float32(jnp.inf), vals)
    d_ref[...] = jnp.concatenate(dl, axis=1)
    i_ref[...] = jnp.concatenate(il, axis=1)


def _tc3(cand, gid):
    return pl.pallas_call(
        _tc3_body,
        grid=(QBLK,),
        in_specs=[
            pl.BlockSpec((QB, KTOP * G), lambda i: (i, 0)),
            pl.BlockSpec((QB, KTOP), lambda i: (i, 0)),
        ],
        out_specs=[
            pl.BlockSpec((QB, KTOP), lambda i: (i, 0)),
            pl.BlockSpec((QB, KTOP), lambda i: (i, 0)),
        ],
        out_shape=[
            jax.ShapeDtypeStruct((Q, KTOP), jnp.float32),
            jax.ShapeDtypeStruct((Q, KTOP), jnp.int32),
        ],
        interpret=_INTERPRET,
    )(cand, gid)


def kernel(queries, keys, k):
    table, gid, rowid = _tc1(queries, keys)
    cand3 = _gather_candidates(
        table.reshape(NG * Q, G), rowid.reshape(IDX_ROWS, 128))
    dists, idx = _tc3(cand3.reshape(Q, KTOP * G), gid)
    shift = (jnp.asarray(k) - KTOP).astype(jnp.float32)
    return (dists + shift, idx)
